# Initial kernel scaffold; baseline (speedup 1.0000x reference)
import jax
import jax.numpy as jnp
from jax.experimental import pallas as pl


def kernel(lit_idx, clause_idx, var_graph_id, clause_graph_id, vq_params, lit_params, clause_params, gate_params, out_params):
    raise NotImplementedError("write your pallas kernel here")


# speedup vs baseline: 1.9143x; 1.9143x over previous
import functools

import jax
import jax.numpy as jnp
from jax import lax
from jax.experimental import pallas as pl
from jax.experimental.pallas import tpu as pltpu
from jax.experimental.pallas import tpu_sc as plsc

NV = 10000
NC = 42000
NG = 16
E = 126000
F = 128
Q = 64
ROUNDS = 4
EPS = 1e-10
B = 128
EPAD = ((E + B - 1) // B) * B


def _softplus(x):
    return jnp.log1p(jnp.exp(-jnp.abs(x))) + jnp.maximum(x, 0.0)


def _mm(x, w):
    return lax.dot_general(x, w, (((1,), (0,)), ((), ())),
                           preferred_element_type=jnp.float32)


def _build_edge_pass(r_src, r_dst, d, n_chunks, n_tables):
    cr = _cr8(r_dst, n_chunks)
    r_out = n_chunks * cr
    rpt = 32 * (-(-(cr + 1) // 512))
    zr = rpt // 4
    crp = rpt * 16
    opt = 8 * (-(-cr // 128))
    rem = cr - 15 * opt
    per_core = n_chunks // 2
    rn = n_chunks * 256

    mesh = plsc.VectorSubcoreMesh(core_axis_name="c", subcore_axis_name="s")
    out_type = [jax.ShapeDtypeStruct((r_out, d), jnp.float32)] * n_tables
    scratch = [pltpu.VMEM((B,), jnp.int32),
               pltpu.VMEM((B,), jnp.int32),
               pltpu.VMEM((B, d), jnp.float32),
               pltpu.VMEM((rn,), jnp.int32)]
    scratch += [pltpu.VMEM_SHARED((crp, d), jnp.float32) for _ in range(n_tables)]

    @functools.partial(pl.kernel, mesh=mesh, out_type=out_type,
                       scratch_types=scratch,
                       compiler_params=pltpu.CompilerParams(
                           use_tc_tiling_on_sc=False))
    def edge_pass(*refs):
        srcs = refs[:n_tables]
        idxh, ldsth, rngh, zh = refs[n_tables:n_tables + 4]
        outs = refs[n_tables + 4:2 * n_tables + 4]
        idx_v, ldst_v, rows_v, rng_v = refs[2 * n_tables + 4:2 * n_tables + 8]
        shs = refs[2 * n_tables + 8:]
        c = lax.axis_index("c")
        s = lax.axis_index("s")
        pltpu.sync_copy(rngh, rng_v)
        for j in range(per_core):
            k = c * per_core + j
            base_row = k * cr
            for z in range(4):
                off = pl.multiple_of(s * rpt + z * zr, 8)
                for sh in shs:
                    pltpu.sync_copy(zh, sh.at[pl.ds(off, zr)])
            plsc.subcore_barrier()
            ent = (k * 16 + s) * 16
            rv = rng_v[pl.ds(pl.multiple_of(ent, 16), 16)]
            st = rv[0]
            nb = rv[1]

            def bat(i, carry):
                eb = pl.multiple_of(st + i * B, B)
                pltpu.sync_copy(idxh.at[pl.ds(eb, B)], idx_v)
                lb = pl.multiple_of(k * EPAD + eb, B)
                pltpu.sync_copy(ldsth.at[pl.ds(lb, B)], ldst_v)
                for t in range(n_tables):
                    pltpu.sync_copy(srcs[t].at[idx_v], rows_v)
                    pltpu.sync_copy(rows_v, shs[t].at[ldst_v], add=True)
                return carry

            lax.fori_loop(0, nb, bat, 0)
            plsc.subcore_barrier()

            @pl.when(s < 15)
            def _():
                src0 = pl.multiple_of(s * opt, 8)
                dst0 = pl.multiple_of(base_row + s * opt, 8)
                for t in range(n_tables):
                    pltpu.sync_copy(shs[t].at[pl.ds(src0, opt)],
                                    outs[t].at[pl.ds(dst0, opt)])

            @pl.when(s == 15)
            def _():
                dst0 = pl.multiple_of(base_row + 15 * opt, 8)
                for t in range(n_tables):
                    pltpu.sync_copy(shs[t].at[pl.ds(15 * opt, rem)],
                                    outs[t].at[pl.ds(dst0, rem)])

            if j + 1 < per_core:
                plsc.subcore_barrier()

    def run(tables, idx, ldst, ranges):
        zeros = jnp.zeros((zr, d), jnp.float32)
        res = edge_pass(*tables, idx, ldst.reshape(-1), ranges, zeros)
        res = tuple(res) if isinstance(res, (list, tuple)) else (res,)
        return tuple(o[:r_dst] for o in res)

    return run


def _cr8(r_dst, n_chunks):
    return 8 * (-(-r_dst // (8 * n_chunks)))


def _prep(key_sorted, n_dst, n_chunks):
    cr = _cr8(n_dst, n_chunks)
    edges = jnp.arange(n_chunks + 1, dtype=jnp.int32) * cr
    bounds = jnp.searchsorted(key_sorted, edges, side="left").astype(jnp.int32)
    b0 = bounds[:-1] // B
    b1 = (bounds[1:] + B - 1) // B
    nb = jnp.maximum(b1 - b0, 0)
    qt, rt = jnp.divmod(nb, 16)
    t = jnp.arange(16, dtype=jnp.int32)
    startb = b0[:, None] + t[None, :] * qt[:, None] + jnp.minimum(t[None, :], rt[:, None])
    cnt = qt[:, None] + (t[None, :] < rt[:, None]).astype(jnp.int32)
    ranges = jnp.zeros((n_chunks * 16, 16), jnp.int32)
    ranges = ranges.at[:, 0].set(startb.reshape(-1) * B)
    ranges = ranges.at[:, 1].set(cnt.reshape(-1))
    ranges = ranges.reshape(-1).astype(jnp.int32)
    rel = key_sorted[None, :] - (jnp.arange(n_chunks, dtype=jnp.int32) * cr)[:, None]
    ldst = jnp.where((rel >= 0) & (rel < cr), rel, cr).astype(jnp.int32)
    return ranges, ldst


RB = NV // NG
CB = NC // NG


def _pair_norm_res(x, prev, beta):
    mean = jnp.mean(x, axis=0, keepdims=True)
    xc = x - mean
    m2 = jnp.mean(xc * xc)
    scale = lax.rsqrt(m2 + 1e-6)
    return xc * scale * 0.25 + beta * prev


def _tc1_body(v_ref, nz_ref, w1_ref, b1_ref, w2_ref, b2_ref, w3_ref, b3_ref,
              l1_ref, c1_ref, l2_ref, c2_ref, l3_ref, c3_ref,
              q_ref, sp_ref, msg_ref):
    x = v_ref[0]
    w1 = w1_ref[...]
    h = jnp.maximum(_mm(x, w1[:F]) + _mm(nz_ref[0], w1[F:]) + b1_ref[...], 0.0)
    h = jnp.maximum(_mm(h, w2_ref[...]) + b2_ref[...], 0.0)
    q = _mm(h, w3_ref[...]) + b3_ref[...]
    q_ref[0] = q
    sp_ref[0, 0] = _softplus(q)
    sp_ref[1, 0] = _softplus(-q)
    m = jnp.maximum(_mm(x, l1_ref[...]) + c1_ref[...], 0.0)
    m = jnp.maximum(_mm(m, l2_ref[...]) + c2_ref[...], 0.0)
    m = _mm(m, l3_ref[...]) + c3_ref[...]
    msg_ref[0, 0] = m[:, :Q]
    msg_ref[1, 0] = m[:, Q:]


def _tc2_body(s_ref, msg_ref, cs_ref, dg_ref, w1_ref, b1_ref, w2_ref, b2_ref,
              w3_ref, b3_ref, cl_ref, vla_ref, ncs_ref):
    S = s_ref[0]
    cl = jnp.exp(-S)
    rdw = lax.rsqrt(jnp.maximum(dg_ref[0][:, :1], 1.0))
    cm = msg_ref[0] * rdw
    cs = cs_ref[0]
    w1 = w1_ref[...]
    h = jnp.maximum(_mm(cs, w1[:F]) + _mm(cm, w1[F:F + Q])
                    + _mm(4.0 * cl, w1[F + Q:]) + b1_ref[...], 0.0)
    h = jnp.maximum(_mm(h, w2_ref[...]) + b2_ref[...], 0.0)
    cd = _mm(h, w3_ref[...]) + b3_ref[...]
    cl_ref[0] = cl
    vla_ref[0] = cd[:, :Q]
    ncs_ref[0] = _pair_norm_res(cd[:, Q:], cs, 0.1)


def _tc3_body(q_ref, rg_ref, vl_ref, v_ref, dg_ref, g1_ref, gb1_ref, g2_ref,
              gb2_ref, g3_ref, gb3_ref, o1_ref, ob1_ref, o2_ref, ob2_ref,
              o3_ref, ob3_ref, nv_ref, logit_ref, t1_ref):
    q = q_ref[0]
    sig = 1.0 / (1.0 + jnp.exp(-q))
    dpos = dg_ref[0, 0][:, :1]
    dneg = dg_ref[1, 0][:, :1]
    vdw = 4.0 * lax.rsqrt(jnp.maximum(dpos + dneg, 1.0))
    grad = (-sig * rg_ref[0, 0] + (1.0 - sig) * rg_ref[1, 0]) * vdw
    vp = vl_ref[0, 0] * lax.rsqrt(jnp.maximum(dpos, 1.0))
    vn = vl_ref[1, 0] * lax.rsqrt(jnp.maximum(dneg, 1.0))
    x = v_ref[0]
    g1 = g1_ref[...]
    h = jnp.maximum(_mm(grad, g1[:Q]) + _mm(x, g1[Q:Q + F])
                    + _mm(vp, g1[Q + F:Q + F + Q]) + _mm(vn, g1[Q + F + Q:])
                    + gb1_ref[...], 0.0)
    h = jnp.maximum(_mm(h, g2_ref[...]) + gb2_ref[...], 0.0)
    nv = _pair_norm_res(_mm(h, g3_ref[...]) + gb3_ref[...], x, 0.1)
    nv_ref[0] = nv
    t = jnp.maximum(_mm(nv, o1_ref[...]) + ob1_ref[...], 0.0)
    t = jnp.maximum(_mm(t, o2_ref[...]) + ob2_ref[...], 0.0)
    lg = _mm(t, o3_ref[...]) + ob3_ref[...]
    logit_ref[0] = lg
    z = jnp.zeros((RB, 7), jnp.float32)
    t1_ref[0, 0] = jnp.concatenate([_softplus(lg), z], axis=1)
    t1_ref[1, 0] = jnp.concatenate([_softplus(-lg), z], axis=1)


def _tc4_body(s1_ref, out_ref):
    cv = jnp.exp(-s1_ref[0][:, :1])
    pcl = cv * (-jnp.log(jnp.maximum(1.0 - cv + EPS, EPS)))
    out_ref[...] = jnp.full((1, 1, 1), 0.0) + jnp.sqrt(jnp.sum(pcl) + 1e-6)


def _full(shape):
    return pl.BlockSpec(shape, lambda i: (0,) * len(shape))


def _blk3(r, c):
    return pl.BlockSpec((1, r, c), lambda i: (i, 0, 0))


def _blk4(r, c):
    return pl.BlockSpec((2, 1, r, c), lambda i: (0, i, 0, 0))


def _make_tc_kernels(interpret=False):
    wspec = _full
    tc1 = pl.pallas_call(
        _tc1_body,
        grid=(NG,),
        in_specs=[_blk3(RB, F), _blk3(RB, 4),
                  wspec((F + 4, F)), wspec((1, F)), wspec((F, F)), wspec((1, F)),
                  wspec((F, Q)), wspec((1, Q)),
                  wspec((F, 4 * Q)), wspec((1, 4 * Q)), wspec((4 * Q, 4 * Q)),
                  wspec((1, 4 * Q)), wspec((4 * Q, 2 * Q)), wspec((1, 2 * Q))],
        out_specs=[_blk3(RB, Q), _blk4(RB, Q), _blk4(RB, Q)],
        out_shape=[jax.ShapeDtypeStruct((NG, RB, Q), jnp.float32),
                   jax.ShapeDtypeStruct((2, NG, RB, Q), jnp.float32),
                   jax.ShapeDtypeStruct((2, NG, RB, Q), jnp.float32)],
        interpret=interpret)
    tc2 = pl.pallas_call(
        _tc2_body,
        grid=(NG,),
        in_specs=[_blk3(CB, Q), _blk3(CB, Q), _blk3(CB, F), _blk3(CB, Q),
                  wspec((F + 2 * Q, 3 * F)), wspec((1, 3 * F)),
                  wspec((3 * F, 3 * F)), wspec((1, 3 * F)),
                  wspec((3 * F, F + Q)), wspec((1, F + Q))],
        out_specs=[_blk3(CB, Q), _blk3(CB, Q), _blk3(CB, F)],
        out_shape=[jax.ShapeDtypeStruct((NG, CB, Q), jnp.float32),
                   jax.ShapeDtypeStruct((NG, CB, Q), jnp.float32),
                   jax.ShapeDtypeStruct((NG, CB, F), jnp.float32)],
        interpret=interpret)
    tc3 = pl.pallas_call(
        _tc3_body,
        grid=(NG,),
        in_specs=[_blk3(RB, Q), _blk4(RB, Q), _blk4(RB, Q), _blk3(RB, F),
                  _blk4(RB, Q),
                  wspec((Q + F + 2 * Q, 2 * F)), wspec((1, 2 * F)),
                  wspec((2 * F, 2 * F)), wspec((1, 2 * F)),
                  wspec((2 * F, F)), wspec((1, F)),
                  wspec((F, F)), wspec((1, F)), wspec((F, F)), wspec((1, F)),
                  wspec((F, 1)), wspec((1, 1))],
        out_specs=[_blk3(RB, F), _blk3(RB, 1), _blk4(RB, 8)],
        out_shape=[jax.ShapeDtypeStruct((NG, RB, F), jnp.float32),
                   jax.ShapeDtypeStruct((NG, RB, 1), jnp.float32),
                   jax.ShapeDtypeStruct((2, NG, RB, 8), jnp.float32)],
        interpret=interpret)
    tc4 = pl.pallas_call(
        _tc4_body,
        grid=(NG,),
        in_specs=[_blk3(CB, 8)],
        out_specs=pl.BlockSpec((1, 1, 1), lambda i: (i, 0, 0)),
        out_shape=jax.ShapeDtypeStruct((NG, 1, 1), jnp.float32),
        interpret=interpret)
    return tc1, tc2, tc3, tc4


def _zero_state_const(n, f, stddev=0.25):
    onehot = jnp.zeros((n, f), jnp.float32).at[:, 0].set(1.0) - 1.0 / f
    return onehot * (float(f) ** 0.5) * stddev


def kernel(lit_idx, clause_idx, var_graph_id, clause_graph_id, vq_params,
           lit_params, clause_params, gate_params, out_params):
    del var_graph_id, clause_graph_id

    pad = EPAD - E
    order_c = jnp.argsort(clause_idx)
    key_c = jnp.concatenate([clause_idx[order_c],
                             jnp.full((pad,), NC, jnp.int32)])
    gidx_c = jnp.concatenate([lit_idx[order_c], jnp.zeros((pad,), jnp.int32)])
    order_l = jnp.argsort(lit_idx)
    key_l = jnp.concatenate([lit_idx[order_l],
                             jnp.full((pad,), 2 * NV, jnp.int32)])
    gidx_l = jnp.concatenate([clause_idx[order_l], jnp.zeros((pad,), jnp.int32)])
    rng_c4, ldst_c4 = _prep(key_c, NC, 4)
    rng_l4, ldst_l4 = _prep(key_l, 2 * NV, 4)
    rng_c2, ldst_c2 = _prep(key_c, NC, 2)

    pass_a = _build_edge_pass(2 * NV, NC, Q, 4, 2)
    pass_b = _build_edge_pass(NC, 2 * NV, Q, 4, 2)
    pass_loss = _build_edge_pass(2 * NV, NC, 8, 2, 1)
    pass_dc = _build_edge_pass(8, NC, Q, 4, 1)
    pass_dl = _build_edge_pass(8, 2 * NV, Q, 4, 1)

    ones8 = jnp.ones((8, Q), jnp.float32)
    zero_idx = jnp.zeros((EPAD,), jnp.int32)
    (deg_c,) = pass_dc([ones8], zero_idx, ldst_c4, rng_c4)
    (deg_l,) = pass_dl([ones8], zero_idx, ldst_l4, rng_l4)
    deg_c3 = deg_c.reshape(NG, CB, Q)
    deg_l4 = deg_l.reshape(2, NG, RB, Q)

    tc1, tc2, tc3, tc4 = _make_tc_kernels()

    def rb(b):
        return b.reshape(1, -1)

    w_tc1 = (vq_params[0][0], rb(vq_params[0][1]), vq_params[1][0],
             rb(vq_params[1][1]), vq_params[2][0], rb(vq_params[2][1]),
             lit_params[0][0], rb(lit_params[0][1]), lit_params[1][0],
             rb(lit_params[1][1]), lit_params[2][0], rb(lit_params[2][1]))
    w_tc2 = (clause_params[0][0], rb(clause_params[0][1]), clause_params[1][0],
             rb(clause_params[1][1]), clause_params[2][0], rb(clause_params[2][1]))
    w_tc3 = (gate_params[0][0], rb(gate_params[0][1]), gate_params[1][0],
             rb(gate_params[1][1]), gate_params[2][0], rb(gate_params[2][1]),
             out_params[0][0], rb(out_params[0][1]), out_params[1][0],
             rb(out_params[1][1]), out_params[2][0], rb(out_params[2][1]))

    variables = _zero_state_const(NV, F).reshape(NG, RB, F)
    clause_state = _zero_state_const(NC, F).reshape(NG, CB, F)
    losses = []
    logits = jnp.zeros((NG, RB, 1), jnp.float32)
    for step in range(ROUNDS):
        noise = jax.random.normal(jax.random.fold_in(jax.random.key(1), step),
                                  (NV, 4), jnp.float32).reshape(NG, RB, 4)
        q, sp2, msg2 = tc1(variables, noise, *w_tc1)
        S, Msg = pass_a([sp2.reshape(2 * NV, Q), msg2.reshape(2 * NV, Q)],
                        gidx_c, ldst_c4, rng_c4)
        cl, vla, clause_state = tc2(S.reshape(NG, CB, Q), Msg.reshape(NG, CB, Q),
                                    clause_state, deg_c3, *w_tc2)
        Rg, VL = pass_b([cl.reshape(NC, Q), vla.reshape(NC, Q)],
                        gidx_l, ldst_l4, rng_l4)
        variables, logits, T1 = tc3(q, Rg.reshape(2, NG, RB, Q),
                                    VL.reshape(2, NG, RB, Q), variables, deg_l4,
                                    *w_tc3)
        (S1,) = pass_loss([T1.reshape(2 * NV, 8)], gidx_c, ldst_c2, rng_c2)
        losses.append(jnp.sum(tc4(S1.reshape(NG, CB, 8))))
    unsupervised_loss = sum(losses) / float(ROUNDS)
    return logits.reshape(NV, 1), unsupervised_loss


# speedup vs baseline: 2.0745x; 1.0837x over previous
import functools

import jax
import jax.numpy as jnp
from jax import lax
from jax.experimental import pallas as pl
from jax.experimental.pallas import tpu as pltpu
from jax.experimental.pallas import tpu_sc as plsc

NV = 10000
NC = 42000
NG = 16
E = 126000
F = 128
Q = 64
ROUNDS = 4
EPS = 1e-10
B = 128
EPAD = ((E + B - 1) // B) * B


def _softplus(x):
    return jnp.log1p(jnp.exp(-jnp.abs(x))) + jnp.maximum(x, 0.0)


def _mm(x, w):
    return lax.dot_general(x, w, (((1,), (0,)), ((), ())),
                           preferred_element_type=jnp.float32)


def _build_edge_pass(r_src, r_dst, d, n_chunks, n_tables):
    cr = _cr8(r_dst, n_chunks)
    r_out = n_chunks * cr
    rpt = 32 * (-(-(cr + 1) // 512))
    zr = rpt // 4
    crp = rpt * 16
    opt = 8 * (-(-cr // 128))
    rem = cr - 15 * opt
    per_core = n_chunks // 2
    rn = n_chunks * 256

    mesh = plsc.VectorSubcoreMesh(core_axis_name="c", subcore_axis_name="s")
    out_type = [jax.ShapeDtypeStruct((r_out, d), jnp.float32)] * n_tables
    scratch = [pltpu.VMEM((B,), jnp.int32),
               pltpu.VMEM((B,), jnp.int32),
               pltpu.VMEM((B, d), jnp.float32),
               pltpu.VMEM((rn,), jnp.int32)]
    scratch += [pltpu.VMEM_SHARED((crp, d), jnp.float32) for _ in range(n_tables)]

    @functools.partial(pl.kernel, mesh=mesh, out_type=out_type,
                       scratch_types=scratch,
                       compiler_params=pltpu.CompilerParams(
                           use_tc_tiling_on_sc=False))
    def edge_pass(*refs):
        srcs = refs[:n_tables]
        idxh, ldsth, rngh, zh = refs[n_tables:n_tables + 4]
        outs = refs[n_tables + 4:2 * n_tables + 4]
        idx_v, ldst_v, rows_v, rng_v = refs[2 * n_tables + 4:2 * n_tables + 8]
        shs = refs[2 * n_tables + 8:]
        c = lax.axis_index("c")
        s = lax.axis_index("s")
        pltpu.sync_copy(rngh, rng_v)
        for j in range(per_core):
            k = c * per_core + j
            base_row = k * cr
            for z in range(4):
                off = pl.multiple_of(s * rpt + z * zr, 8)
                for sh in shs:
                    pltpu.sync_copy(zh, sh.at[pl.ds(off, zr)])
            plsc.subcore_barrier()
            ent = (k * 16 + s) * 16
            rv = rng_v[pl.ds(pl.multiple_of(ent, 16), 16)]
            st = rv[0]
            nb = rv[1]

            def bat(i, carry):
                eb = pl.multiple_of(st + i * B, B)
                pltpu.sync_copy(idxh.at[pl.ds(eb, B)], idx_v)
                lb = pl.multiple_of(k * EPAD + eb, B)
                pltpu.sync_copy(ldsth.at[pl.ds(lb, B)], ldst_v)
                for t in range(n_tables):
                    pltpu.sync_copy(srcs[t].at[idx_v], rows_v)
                    pltpu.sync_copy(rows_v, shs[t].at[ldst_v], add=True)
                return carry

            lax.fori_loop(0, nb, bat, 0)
            plsc.subcore_barrier()

            @pl.when(s < 15)
            def _():
                src0 = pl.multiple_of(s * opt, 8)
                dst0 = pl.multiple_of(base_row + s * opt, 8)
                for t in range(n_tables):
                    pltpu.sync_copy(shs[t].at[pl.ds(src0, opt)],
                                    outs[t].at[pl.ds(dst0, opt)])

            @pl.when(s == 15)
            def _():
                dst0 = pl.multiple_of(base_row + 15 * opt, 8)
                for t in range(n_tables):
                    pltpu.sync_copy(shs[t].at[pl.ds(15 * opt, rem)],
                                    outs[t].at[pl.ds(dst0, rem)])

            if j + 1 < per_core:
                plsc.subcore_barrier()

    def run(tables, idx, ldst, ranges):
        zeros = jnp.zeros((zr, d), jnp.float32)
        res = edge_pass(*tables, idx, ldst.reshape(-1), ranges, zeros)
        res = tuple(res) if isinstance(res, (list, tuple)) else (res,)
        return tuple(o[:r_dst] for o in res)

    return run


def _cr8(r_dst, n_chunks):
    return 8 * (-(-r_dst // (8 * n_chunks)))


def _prep(key_sorted, n_dst, n_chunks):
    cr = _cr8(n_dst, n_chunks)
    edges = jnp.arange(n_chunks + 1, dtype=jnp.int32) * cr
    bounds = jnp.searchsorted(key_sorted, edges, side="left").astype(jnp.int32)
    b0 = bounds[:-1] // B
    b1 = (bounds[1:] + B - 1) // B
    nb = jnp.maximum(b1 - b0, 0)
    qt, rt = jnp.divmod(nb, 16)
    t = jnp.arange(16, dtype=jnp.int32)
    startb = b0[:, None] + t[None, :] * qt[:, None] + jnp.minimum(t[None, :], rt[:, None])
    cnt = qt[:, None] + (t[None, :] < rt[:, None]).astype(jnp.int32)
    ranges = jnp.zeros((n_chunks * 16, 16), jnp.int32)
    ranges = ranges.at[:, 0].set(startb.reshape(-1) * B)
    ranges = ranges.at[:, 1].set(cnt.reshape(-1))
    ranges = ranges.reshape(-1).astype(jnp.int32)
    rel = key_sorted[None, :] - (jnp.arange(n_chunks, dtype=jnp.int32) * cr)[:, None]
    ldst = jnp.where((rel >= 0) & (rel < cr), rel, cr).astype(jnp.int32)
    return ranges, ldst


RB = NV // NG
CB = NC // NG


def _pair_norm_res(x, prev, beta):
    mean = jnp.mean(x, axis=0, keepdims=True)
    xc = x - mean
    m2 = jnp.mean(xc * xc)
    scale = lax.rsqrt(m2 + 1e-6)
    return xc * scale * 0.25 + beta * prev


def _tc1_body(v_ref, nz_ref, w1_ref, b1_ref, w2_ref, b2_ref, w3_ref, b3_ref,
              l1_ref, c1_ref, l2_ref, c2_ref, l3_ref, c3_ref,
              q_ref, sm_ref):
    x = v_ref[0]
    w1 = w1_ref[...]
    h = jnp.maximum(_mm(x, w1[:F]) + _mm(nz_ref[0], w1[F:]) + b1_ref[...], 0.0)
    h = jnp.maximum(_mm(h, w2_ref[...]) + b2_ref[...], 0.0)
    q = _mm(h, w3_ref[...]) + b3_ref[...]
    q_ref[0] = q
    m = jnp.maximum(_mm(x, l1_ref[...]) + c1_ref[...], 0.0)
    m = jnp.maximum(_mm(m, l2_ref[...]) + c2_ref[...], 0.0)
    m = _mm(m, l3_ref[...]) + c3_ref[...]
    sm_ref[0, 0] = jnp.concatenate([_softplus(q), m[:, :Q]], axis=1)
    sm_ref[1, 0] = jnp.concatenate([_softplus(-q), m[:, Q:]], axis=1)


def _tc2_body(sm_ref, cs_ref, dg_ref, w1_ref, b1_ref, w2_ref, b2_ref,
              w3_ref, b3_ref, cv_ref, ncs_ref):
    sm = sm_ref[0]
    cl = jnp.exp(-sm[:, :Q])
    rdw = lax.rsqrt(jnp.maximum(dg_ref[0][:, :1], 1.0))
    cm = sm[:, Q:] * rdw
    cs = cs_ref[0]
    w1 = w1_ref[...]
    h = jnp.maximum(_mm(cs, w1[:F]) + _mm(cm, w1[F:F + Q])
                    + _mm(4.0 * cl, w1[F + Q:]) + b1_ref[...], 0.0)
    h = jnp.maximum(_mm(h, w2_ref[...]) + b2_ref[...], 0.0)
    cd = _mm(h, w3_ref[...]) + b3_ref[...]
    cv_ref[0] = jnp.concatenate([cl, cd[:, :Q]], axis=1)
    ncs_ref[0] = _pair_norm_res(cd[:, Q:], cs, 0.1)


def _tc3_body(q_ref, rgvl_ref, v_ref, dg_ref, g1_ref, gb1_ref, g2_ref,
              gb2_ref, g3_ref, gb3_ref, o1_ref, ob1_ref, o2_ref, ob2_ref,
              o3_ref, ob3_ref, nv_ref, logit_ref, t1_ref):
    q = q_ref[0]
    sig = 1.0 / (1.0 + jnp.exp(-q))
    dpos = dg_ref[0, 0][:, :1]
    dneg = dg_ref[1, 0][:, :1]
    vdw = 4.0 * lax.rsqrt(jnp.maximum(dpos + dneg, 1.0))
    rv_p = rgvl_ref[0, 0]
    rv_n = rgvl_ref[1, 0]
    grad = (-sig * rv_p[:, :Q] + (1.0 - sig) * rv_n[:, :Q]) * vdw
    vp = rv_p[:, Q:] * lax.rsqrt(jnp.maximum(dpos, 1.0))
    vn = rv_n[:, Q:] * lax.rsqrt(jnp.maximum(dneg, 1.0))
    x = v_ref[0]
    g1 = g1_ref[...]
    h = jnp.maximum(_mm(grad, g1[:Q]) + _mm(x, g1[Q:Q + F])
                    + _mm(vp, g1[Q + F:Q + F + Q]) + _mm(vn, g1[Q + F + Q:])
                    + gb1_ref[...], 0.0)
    h = jnp.maximum(_mm(h, g2_ref[...]) + gb2_ref[...], 0.0)
    nv = _pair_norm_res(_mm(h, g3_ref[...]) + gb3_ref[...], x, 0.1)
    nv_ref[0] = nv
    t = jnp.maximum(_mm(nv, o1_ref[...]) + ob1_ref[...], 0.0)
    t = jnp.maximum(_mm(t, o2_ref[...]) + ob2_ref[...], 0.0)
    lg = _mm(t, o3_ref[...]) + ob3_ref[...]
    logit_ref[0] = lg
    z = jnp.zeros((RB, 7), jnp.float32)
    t1_ref[0, 0] = jnp.concatenate([_softplus(lg), z], axis=1)
    t1_ref[1, 0] = jnp.concatenate([_softplus(-lg), z], axis=1)


def _tc4_body(s1_ref, out_ref):
    cv = jnp.exp(-s1_ref[0][:, :1])
    pcl = cv * (-jnp.log(jnp.maximum(1.0 - cv + EPS, EPS)))
    out_ref[...] = jnp.full((1, 1, 1), 0.0) + jnp.sqrt(jnp.sum(pcl) + 1e-6)


def _full(shape):
    return pl.BlockSpec(shape, lambda i: (0,) * len(shape))


def _blk3(r, c):
    return pl.BlockSpec((1, r, c), lambda i: (i, 0, 0))


def _blk4(r, c):
    return pl.BlockSpec((2, 1, r, c), lambda i: (0, i, 0, 0))


def _make_tc_kernels(interpret=False):
    wspec = _full
    tc1 = pl.pallas_call(
        _tc1_body,
        grid=(NG,),
        in_specs=[_blk3(RB, F), _blk3(RB, 4),
                  wspec((F + 4, F)), wspec((1, F)), wspec((F, F)), wspec((1, F)),
                  wspec((F, Q)), wspec((1, Q)),
                  wspec((F, 4 * Q)), wspec((1, 4 * Q)), wspec((4 * Q, 4 * Q)),
                  wspec((1, 4 * Q)), wspec((4 * Q, 2 * Q)), wspec((1, 2 * Q))],
        out_specs=[_blk3(RB, Q), _blk4(RB, 2 * Q)],
        out_shape=[jax.ShapeDtypeStruct((NG, RB, Q), jnp.float32),
                   jax.ShapeDtypeStruct((2, NG, RB, 2 * Q), jnp.float32)],
        interpret=interpret)
    tc2 = pl.pallas_call(
        _tc2_body,
        grid=(NG,),
        in_specs=[_blk3(CB, 2 * Q), _blk3(CB, F), _blk3(CB, Q),
                  wspec((F + 2 * Q, 3 * F)), wspec((1, 3 * F)),
                  wspec((3 * F, 3 * F)), wspec((1, 3 * F)),
                  wspec((3 * F, F + Q)), wspec((1, F + Q))],
        out_specs=[_blk3(CB, 2 * Q), _blk3(CB, F)],
        out_shape=[jax.ShapeDtypeStruct((NG, CB, 2 * Q), jnp.float32),
                   jax.ShapeDtypeStruct((NG, CB, F), jnp.float32)],
        interpret=interpret)
    tc3 = pl.pallas_call(
        _tc3_body,
        grid=(NG,),
        in_specs=[_blk3(RB, Q), _blk4(RB, 2 * Q), _blk3(RB, F),
                  _blk4(RB, Q),
                  wspec((Q + F + 2 * Q, 2 * F)), wspec((1, 2 * F)),
                  wspec((2 * F, 2 * F)), wspec((1, 2 * F)),
                  wspec((2 * F, F)), wspec((1, F)),
                  wspec((F, F)), wspec((1, F)), wspec((F, F)), wspec((1, F)),
                  wspec((F, 1)), wspec((1, 1))],
        out_specs=[_blk3(RB, F), _blk3(RB, 1), _blk4(RB, 8)],
        out_shape=[jax.ShapeDtypeStruct((NG, RB, F), jnp.float32),
                   jax.ShapeDtypeStruct((NG, RB, 1), jnp.float32),
                   jax.ShapeDtypeStruct((2, NG, RB, 8), jnp.float32)],
        interpret=interpret)
    tc4 = pl.pallas_call(
        _tc4_body,
        grid=(NG,),
        in_specs=[_blk3(CB, 8)],
        out_specs=pl.BlockSpec((1, 1, 1), lambda i: (i, 0, 0)),
        out_shape=jax.ShapeDtypeStruct((NG, 1, 1), jnp.float32),
        interpret=interpret)
    return tc1, tc2, tc3, tc4


def _zero_state_const(n, f, stddev=0.25):
    onehot = jnp.zeros((n, f), jnp.float32).at[:, 0].set(1.0) - 1.0 / f
    return onehot * (float(f) ** 0.5) * stddev


def kernel(lit_idx, clause_idx, var_graph_id, clause_graph_id, vq_params,
           lit_params, clause_params, gate_params, out_params):
    del var_graph_id, clause_graph_id

    pad = EPAD - E
    order_c = jnp.argsort(clause_idx)
    key_c = jnp.concatenate([clause_idx[order_c],
                             jnp.full((pad,), NC, jnp.int32)])
    gidx_c = jnp.concatenate([lit_idx[order_c], jnp.zeros((pad,), jnp.int32)])
    order_l = jnp.argsort(lit_idx)
    key_l = jnp.concatenate([lit_idx[order_l],
                             jnp.full((pad,), 2 * NV, jnp.int32)])
    gidx_l = jnp.concatenate([clause_idx[order_l], jnp.zeros((pad,), jnp.int32)])
    rng_c4, ldst_c4 = _prep(key_c, NC, 4)
    rng_l4, ldst_l4 = _prep(key_l, 2 * NV, 4)
    rng_c2, ldst_c2 = _prep(key_c, NC, 2)

    pass_a = _build_edge_pass(2 * NV, NC, 2 * Q, 4, 1)
    pass_b = _build_edge_pass(NC, 2 * NV, 2 * Q, 4, 1)
    pass_loss = _build_edge_pass(2 * NV, NC, 8, 2, 1)
    pass_dc = _build_edge_pass(8, NC, Q, 4, 1)
    pass_dl = _build_edge_pass(8, 2 * NV, Q, 4, 1)

    ones8 = jnp.ones((8, Q), jnp.float32)
    zero_idx = jnp.zeros((EPAD,), jnp.int32)
    (deg_c,) = pass_dc([ones8], zero_idx, ldst_c4, rng_c4)
    (deg_l,) = pass_dl([ones8], zero_idx, ldst_l4, rng_l4)
    deg_c3 = deg_c.reshape(NG, CB, Q)
    deg_l4 = deg_l.reshape(2, NG, RB, Q)

    tc1, tc2, tc3, tc4 = _make_tc_kernels()

    def rb(b):
        return b.reshape(1, -1)

    w_tc1 = (vq_params[0][0], rb(vq_params[0][1]), vq_params[1][0],
             rb(vq_params[1][1]), vq_params[2][0], rb(vq_params[2][1]),
             lit_params[0][0], rb(lit_params[0][1]), lit_params[1][0],
             rb(lit_params[1][1]), lit_params[2][0], rb(lit_params[2][1]))
    w_tc2 = (clause_params[0][0], rb(clause_params[0][1]), clause_params[1][0],
             rb(clause_params[1][1]), clause_params[2][0], rb(clause_params[2][1]))
    w_tc3 = (gate_params[0][0], rb(gate_params[0][1]), gate_params[1][0],
             rb(gate_params[1][1]), gate_params[2][0], rb(gate_params[2][1]),
             out_params[0][0], rb(out_params[0][1]), out_params[1][0],
             rb(out_params[1][1]), out_params[2][0], rb(out_params[2][1]))

    variables = _zero_state_const(NV, F).reshape(NG, RB, F)
    clause_state = _zero_state_const(NC, F).reshape(NG, CB, F)
    losses = []
    logits = jnp.zeros((NG, RB, 1), jnp.float32)
    for step in range(ROUNDS):
        noise = jax.random.normal(jax.random.fold_in(jax.random.key(1), step),
                                  (NV, 4), jnp.float32).reshape(NG, RB, 4)
        q, sm2 = tc1(variables, noise, *w_tc1)
        (SM,) = pass_a([sm2.reshape(2 * NV, 2 * Q)], gidx_c, ldst_c4, rng_c4)
        cv, clause_state = tc2(SM.reshape(NG, CB, 2 * Q),
                               clause_state, deg_c3, *w_tc2)
        (RV,) = pass_b([cv.reshape(NC, 2 * Q)], gidx_l, ldst_l4, rng_l4)
        variables, logits, T1 = tc3(q, RV.reshape(2, NG, RB, 2 * Q),
                                    variables, deg_l4, *w_tc3)
        (S1,) = pass_loss([T1.reshape(2 * NV, 8)], gidx_c, ldst_c2, rng_c2)
        losses.append(jnp.sum(tc4(S1.reshape(NG, CB, 8))))
    unsupervised_loss = sum(losses) / float(ROUNDS)
    return logits.reshape(NV, 1), unsupervised_loss


# speedup vs baseline: 2.1551x; 1.0389x over previous
import functools

import jax
import jax.numpy as jnp
from jax import lax
from jax.experimental import pallas as pl
from jax.experimental.pallas import tpu as pltpu
from jax.experimental.pallas import tpu_sc as plsc

NV = 10000
NC = 42000
NG = 16
E = 126000
F = 128
Q = 64
ROUNDS = 4
EPS = 1e-10
B = 128
EPAD = ((E + B - 1) // B) * B


def _softplus(x):
    return jnp.log1p(jnp.exp(-jnp.abs(x))) + jnp.maximum(x, 0.0)


def _mm(x, w):
    return lax.dot_general(x, w, (((1,), (0,)), ((), ())),
                           preferred_element_type=jnp.float32)


def _build_edge_pass(r_src, r_dst, d, n_chunks, n_tables):
    cr = _cr8(r_dst, n_chunks)
    r_out = n_chunks * cr
    rpt = 32 * (-(-(cr + 1) // 512))
    crp = rpt * 16
    opt = 8 * (-(-cr // 128))
    rem = cr - 15 * opt
    per_core = n_chunks // 2
    rn = n_chunks * 256

    mesh = plsc.VectorSubcoreMesh(core_axis_name="c", subcore_axis_name="s")
    out_type = [jax.ShapeDtypeStruct((r_out, d), jnp.float32)] * n_tables
    scratch = [pltpu.VMEM((2 * B,), jnp.int32),
               pltpu.VMEM((B, d), jnp.float32),
               pltpu.VMEM((rn,), jnp.int32)]
    scratch += [pltpu.VMEM_SHARED((crp, d), jnp.float32) for _ in range(n_tables)]

    @functools.partial(pl.kernel, mesh=mesh, out_type=out_type,
                       scratch_types=scratch,
                       compiler_params=pltpu.CompilerParams(
                           use_tc_tiling_on_sc=False))
    def edge_pass(*refs):
        srcs = refs[:n_tables]
        combh, rngh, zh = refs[n_tables:n_tables + 3]
        outs = refs[n_tables + 3:2 * n_tables + 3]
        cv, rows_v, rng_v = refs[2 * n_tables + 3:2 * n_tables + 6]
        shs = refs[2 * n_tables + 6:]
        c = lax.axis_index("c")
        s = lax.axis_index("s")
        pltpu.sync_copy(rngh, rng_v)
        for j in range(per_core):
            k = c * per_core + j
            base_row = k * cr
            off = pl.multiple_of(s * rpt, 8)
            for sh in shs:
                pltpu.sync_copy(zh, sh.at[pl.ds(off, rpt)])
            plsc.subcore_barrier()
            ent = (k * 16 + s) * 16
            rv = rng_v[pl.ds(pl.multiple_of(ent, 16), 16)]
            st = rv[0]
            nb = rv[1]

            def bat(i, carry):
                eb = pl.multiple_of(st + i * B, B)
                cb = pl.multiple_of(2 * (k * EPAD + eb), 2 * B)
                pltpu.sync_copy(combh.at[pl.ds(cb, 2 * B)], cv)
                for t in range(n_tables):
                    pltpu.sync_copy(srcs[t].at[cv.at[pl.ds(0, B)]], rows_v)
                    pltpu.sync_copy(rows_v, shs[t].at[cv.at[pl.ds(B, B)]],
                                    add=True)
                return carry

            lax.fori_loop(0, nb, bat, 0)
            plsc.subcore_barrier()

            @pl.when(s < 15)
            def _():
                src0 = pl.multiple_of(s * opt, 8)
                dst0 = pl.multiple_of(base_row + s * opt, 8)
                for t in range(n_tables):
                    pltpu.sync_copy(shs[t].at[pl.ds(src0, opt)],
                                    outs[t].at[pl.ds(dst0, opt)])

            @pl.when(s == 15)
            def _():
                dst0 = pl.multiple_of(base_row + 15 * opt, 8)
                for t in range(n_tables):
                    pltpu.sync_copy(shs[t].at[pl.ds(15 * opt, rem)],
                                    outs[t].at[pl.ds(dst0, rem)])

            if j + 1 < per_core:
                plsc.subcore_barrier()

    def run(tables, comb, ranges):
        zeros = jnp.zeros((rpt, d), jnp.float32)
        res = edge_pass(*tables, comb, ranges, zeros)
        res = tuple(res) if isinstance(res, (list, tuple)) else (res,)
        return tuple(o[:r_dst] for o in res)

    return run


def _cr8(r_dst, n_chunks):
    return 8 * (-(-r_dst // (8 * n_chunks)))


def _prep(key_sorted, n_dst, n_chunks):
    cr = _cr8(n_dst, n_chunks)
    edges = jnp.arange(n_chunks + 1, dtype=jnp.int32) * cr
    bounds = jnp.searchsorted(key_sorted, edges, side="left").astype(jnp.int32)
    b0 = bounds[:-1] // B
    b1 = (bounds[1:] + B - 1) // B
    nb = jnp.maximum(b1 - b0, 0)
    qt, rt = jnp.divmod(nb, 16)
    t = jnp.arange(16, dtype=jnp.int32)
    startb = b0[:, None] + t[None, :] * qt[:, None] + jnp.minimum(t[None, :], rt[:, None])
    cnt = qt[:, None] + (t[None, :] < rt[:, None]).astype(jnp.int32)
    ranges = jnp.zeros((n_chunks * 16, 16), jnp.int32)
    ranges = ranges.at[:, 0].set(startb.reshape(-1) * B)
    ranges = ranges.at[:, 1].set(cnt.reshape(-1))
    ranges = ranges.reshape(-1).astype(jnp.int32)
    rel = key_sorted[None, :] - (jnp.arange(n_chunks, dtype=jnp.int32) * cr)[:, None]
    ldst = jnp.where((rel >= 0) & (rel < cr), rel, cr).astype(jnp.int32)
    return ranges, ldst


def _comb(gidx, ldst):
    nb = EPAD // B
    n_chunks = ldst.shape[0]
    g = jnp.broadcast_to(gidx.reshape(1, nb, 1, B), (n_chunks, nb, 1, B))
    l = ldst.reshape(n_chunks, nb, 1, B)
    return jnp.concatenate([g, l], axis=2).reshape(-1).astype(jnp.int32)


RB = NV // NG
CB = NC // NG


def _pair_norm_res(x, prev, beta):
    mean = jnp.mean(x, axis=0, keepdims=True)
    xc = x - mean
    m2 = jnp.mean(xc * xc)
    scale = lax.rsqrt(m2 + 1e-6)
    return xc * scale * 0.25 + beta * prev


def _tc1_body(v_ref, nz_ref, w1_ref, b1_ref, w2_ref, b2_ref, w3_ref, b3_ref,
              l1_ref, c1_ref, l2_ref, c2_ref, l3_ref, c3_ref,
              q_ref, sm_ref):
    x = v_ref[0]
    w1 = w1_ref[...]
    h = jnp.maximum(_mm(x, w1[:F]) + _mm(nz_ref[0], w1[F:]) + b1_ref[...], 0.0)
    h = jnp.maximum(_mm(h, w2_ref[...]) + b2_ref[...], 0.0)
    q = _mm(h, w3_ref[...]) + b3_ref[...]
    q_ref[0] = q
    m = jnp.maximum(_mm(x, l1_ref[...]) + c1_ref[...], 0.0)
    m = jnp.maximum(_mm(m, l2_ref[...]) + c2_ref[...], 0.0)
    m = _mm(m, l3_ref[...]) + c3_ref[...]
    sm_ref[0, 0] = jnp.concatenate([_softplus(q), m[:, :Q]], axis=1)
    sm_ref[1, 0] = jnp.concatenate([_softplus(-q), m[:, Q:]], axis=1)


def _tc2_body(sm_ref, cs_ref, dg_ref, w1_ref, b1_ref, w2_ref, b2_ref,
              w3_ref, b3_ref, cv_ref, ncs_ref):
    sm = sm_ref[0]
    cl = jnp.exp(-sm[:, :Q])
    rdw = lax.rsqrt(jnp.maximum(dg_ref[0][:, :1], 1.0))
    cm = sm[:, Q:] * rdw
    cs = cs_ref[0]
    w1 = w1_ref[...]
    h = jnp.maximum(_mm(cs, w1[:F]) + _mm(cm, w1[F:F + Q])
                    + _mm(4.0 * cl, w1[F + Q:]) + b1_ref[...], 0.0)
    h = jnp.maximum(_mm(h, w2_ref[...]) + b2_ref[...], 0.0)
    cd = _mm(h, w3_ref[...]) + b3_ref[...]
    cv_ref[0] = jnp.concatenate([cl, cd[:, :Q]], axis=1)
    ncs_ref[0] = _pair_norm_res(cd[:, Q:], cs, 0.1)


def _tc3_body(q_ref, rgvl_ref, v_ref, dg_ref, g1_ref, gb1_ref, g2_ref,
              gb2_ref, g3_ref, gb3_ref, o1_ref, ob1_ref, o2_ref, ob2_ref,
              o3_ref, ob3_ref, nv_ref, logit_ref, t1_ref):
    q = q_ref[0]
    sig = 1.0 / (1.0 + jnp.exp(-q))
    dpos = dg_ref[0, 0][:, :1]
    dneg = dg_ref[1, 0][:, :1]
    vdw = 4.0 * lax.rsqrt(jnp.maximum(dpos + dneg, 1.0))
    rv_p = rgvl_ref[0, 0]
    rv_n = rgvl_ref[1, 0]
    grad = (-sig * rv_p[:, :Q] + (1.0 - sig) * rv_n[:, :Q]) * vdw
    vp = rv_p[:, Q:] * lax.rsqrt(jnp.maximum(dpos, 1.0))
    vn = rv_n[:, Q:] * lax.rsqrt(jnp.maximum(dneg, 1.0))
    x = v_ref[0]
    g1 = g1_ref[...]
    h = jnp.maximum(_mm(grad, g1[:Q]) + _mm(x, g1[Q:Q + F])
                    + _mm(vp, g1[Q + F:Q + F + Q]) + _mm(vn, g1[Q + F + Q:])
                    + gb1_ref[...], 0.0)
    h = jnp.maximum(_mm(h, g2_ref[...]) + gb2_ref[...], 0.0)
    nv = _pair_norm_res(_mm(h, g3_ref[...]) + gb3_ref[...], x, 0.1)
    nv_ref[0] = nv
    t = jnp.maximum(_mm(nv, o1_ref[...]) + ob1_ref[...], 0.0)
    t = jnp.maximum(_mm(t, o2_ref[...]) + ob2_ref[...], 0.0)
    lg = _mm(t, o3_ref[...]) + ob3_ref[...]
    logit_ref[0] = lg
    z = jnp.zeros((RB, 7), jnp.float32)
    t1_ref[0, 0] = jnp.concatenate([_softplus(lg), z], axis=1)
    t1_ref[1, 0] = jnp.concatenate([_softplus(-lg), z], axis=1)


def _tc4_body(s1_ref, out_ref):
    cv = jnp.exp(-s1_ref[0][:, :1])
    pcl = cv * (-jnp.log(jnp.maximum(1.0 - cv + EPS, EPS)))
    out_ref[...] = jnp.full((1, 1, 1), 0.0) + jnp.sqrt(jnp.sum(pcl) + 1e-6)


def _full(shape):
    return pl.BlockSpec(shape, lambda i: (0,) * len(shape))


def _blk3(r, c):
    return pl.BlockSpec((1, r, c), lambda i: (i, 0, 0))


def _blk4(r, c):
    return pl.BlockSpec((2, 1, r, c), lambda i: (0, i, 0, 0))


def _make_tc_kernels(interpret=False):
    wspec = _full
    tc1 = pl.pallas_call(
        _tc1_body,
        grid=(NG,),
        in_specs=[_blk3(RB, F), _blk3(RB, 4),
                  wspec((F + 4, F)), wspec((1, F)), wspec((F, F)), wspec((1, F)),
                  wspec((F, Q)), wspec((1, Q)),
                  wspec((F, 4 * Q)), wspec((1, 4 * Q)), wspec((4 * Q, 4 * Q)),
                  wspec((1, 4 * Q)), wspec((4 * Q, 2 * Q)), wspec((1, 2 * Q))],
        out_specs=[_blk3(RB, Q), _blk4(RB, 2 * Q)],
        out_shape=[jax.ShapeDtypeStruct((NG, RB, Q), jnp.float32),
                   jax.ShapeDtypeStruct((2, NG, RB, 2 * Q), jnp.float32)],
        interpret=interpret)
    tc2 = pl.pallas_call(
        _tc2_body,
        grid=(NG,),
        in_specs=[_blk3(CB, 2 * Q), _blk3(CB, F), _blk3(CB, Q),
                  wspec((F + 2 * Q, 3 * F)), wspec((1, 3 * F)),
                  wspec((3 * F, 3 * F)), wspec((1, 3 * F)),
                  wspec((3 * F, F + Q)), wspec((1, F + Q))],
        out_specs=[_blk3(CB, 2 * Q), _blk3(CB, F)],
        out_shape=[jax.ShapeDtypeStruct((NG, CB, 2 * Q), jnp.float32),
                   jax.ShapeDtypeStruct((NG, CB, F), jnp.float32)],
        interpret=interpret)
    tc3 = pl.pallas_call(
        _tc3_body,
        grid=(NG,),
        in_specs=[_blk3(RB, Q), _blk4(RB, 2 * Q), _blk3(RB, F),
                  _blk4(RB, Q),
                  wspec((Q + F + 2 * Q, 2 * F)), wspec((1, 2 * F)),
                  wspec((2 * F, 2 * F)), wspec((1, 2 * F)),
                  wspec((2 * F, F)), wspec((1, F)),
                  wspec((F, F)), wspec((1, F)), wspec((F, F)), wspec((1, F)),
                  wspec((F, 1)), wspec((1, 1))],
        out_specs=[_blk3(RB, F), _blk3(RB, 1), _blk4(RB, 8)],
        out_shape=[jax.ShapeDtypeStruct((NG, RB, F), jnp.float32),
                   jax.ShapeDtypeStruct((NG, RB, 1), jnp.float32),
                   jax.ShapeDtypeStruct((2, NG, RB, 8), jnp.float32)],
        interpret=interpret)
    tc4 = pl.pallas_call(
        _tc4_body,
        grid=(NG,),
        in_specs=[_blk3(CB, 8)],
        out_specs=pl.BlockSpec((1, 1, 1), lambda i: (i, 0, 0)),
        out_shape=jax.ShapeDtypeStruct((NG, 1, 1), jnp.float32),
        interpret=interpret)
    return tc1, tc2, tc3, tc4


def _zero_state_const(n, f, stddev=0.25):
    onehot = jnp.zeros((n, f), jnp.float32).at[:, 0].set(1.0) - 1.0 / f
    return onehot * (float(f) ** 0.5) * stddev


def kernel(lit_idx, clause_idx, var_graph_id, clause_graph_id, vq_params,
           lit_params, clause_params, gate_params, out_params):
    del var_graph_id, clause_graph_id

    pad = EPAD - E
    order_c = jnp.argsort(clause_idx)
    key_c = jnp.concatenate([clause_idx[order_c],
                             jnp.full((pad,), NC, jnp.int32)])
    gidx_c = jnp.concatenate([lit_idx[order_c], jnp.zeros((pad,), jnp.int32)])
    order_l = jnp.argsort(lit_idx)
    key_l = jnp.concatenate([lit_idx[order_l],
                             jnp.full((pad,), 2 * NV, jnp.int32)])
    gidx_l = jnp.concatenate([clause_idx[order_l], jnp.zeros((pad,), jnp.int32)])
    rng_c4, ldst_c4 = _prep(key_c, NC, 4)
    rng_l4, ldst_l4 = _prep(key_l, 2 * NV, 4)
    rng_c2, ldst_c2 = _prep(key_c, NC, 2)
    zero_idx = jnp.zeros((EPAD,), jnp.int32)
    comb_c4 = _comb(gidx_c, ldst_c4)
    comb_l4 = _comb(gidx_l, ldst_l4)
    comb_c2 = _comb(gidx_c, ldst_c2)
    comb_c4z = _comb(zero_idx, ldst_c4)
    comb_l4z = _comb(zero_idx, ldst_l4)

    pass_a = _build_edge_pass(2 * NV, NC, 2 * Q, 4, 1)
    pass_b = _build_edge_pass(NC, 2 * NV, 2 * Q, 4, 1)
    pass_loss = _build_edge_pass(2 * NV, NC, 8, 2, 1)
    pass_dc = _build_edge_pass(8, NC, Q, 4, 1)
    pass_dl = _build_edge_pass(8, 2 * NV, Q, 4, 1)

    ones8 = jnp.ones((8, Q), jnp.float32)
    (deg_c,) = pass_dc([ones8], comb_c4z, rng_c4)
    (deg_l,) = pass_dl([ones8], comb_l4z, rng_l4)
    deg_c3 = deg_c.reshape(NG, CB, Q)
    deg_l4 = deg_l.reshape(2, NG, RB, Q)

    tc1, tc2, tc3, tc4 = _make_tc_kernels()

    def rb(b):
        return b.reshape(1, -1)

    w_tc1 = (vq_params[0][0], rb(vq_params[0][1]), vq_params[1][0],
             rb(vq_params[1][1]), vq_params[2][0], rb(vq_params[2][1]),
             lit_params[0][0], rb(lit_params[0][1]), lit_params[1][0],
             rb(lit_params[1][1]), lit_params[2][0], rb(lit_params[2][1]))
    w_tc2 = (clause_params[0][0], rb(clause_params[0][1]), clause_params[1][0],
             rb(clause_params[1][1]), clause_params[2][0], rb(clause_params[2][1]))
    w_tc3 = (gate_params[0][0], rb(gate_params[0][1]), gate_params[1][0],
             rb(gate_params[1][1]), gate_params[2][0], rb(gate_params[2][1]),
             out_params[0][0], rb(out_params[0][1]), out_params[1][0],
             rb(out_params[1][1]), out_params[2][0], rb(out_params[2][1]))

    variables = _zero_state_const(NV, F).reshape(NG, RB, F)
    clause_state = _zero_state_const(NC, F).reshape(NG, CB, F)
    losses = []
    logits = jnp.zeros((NG, RB, 1), jnp.float32)
    for step in range(ROUNDS):
        noise = jax.random.normal(jax.random.fold_in(jax.random.key(1), step),
                                  (NV, 4), jnp.float32).reshape(NG, RB, 4)
        q, sm2 = tc1(variables, noise, *w_tc1)
        (SM,) = pass_a([sm2.reshape(2 * NV, 2 * Q)], comb_c4, rng_c4)
        cv, clause_state = tc2(SM.reshape(NG, CB, 2 * Q),
                               clause_state, deg_c3, *w_tc2)
        (RV,) = pass_b([cv.reshape(NC, 2 * Q)], comb_l4, rng_l4)
        variables, logits, T1 = tc3(q, RV.reshape(2, NG, RB, 2 * Q),
                                    variables, deg_l4, *w_tc3)
        (S1,) = pass_loss([T1.reshape(2 * NV, 8)], comb_c2, rng_c2)
        losses.append(jnp.sum(tc4(S1.reshape(NG, CB, 8))))
    unsupervised_loss = sum(losses) / float(ROUNDS)
    return logits.reshape(NV, 1), unsupervised_loss
